# Initial kernel scaffold; baseline (speedup 1.0000x reference)
import jax
import jax.numpy as jnp
from jax.experimental import pallas as pl


def kernel(embed, adj_sp_norm, edge_index, edge_weight, trend):
    raise NotImplementedError("write your pallas kernel here")


# speedup vs baseline: 1.6564x; 1.6564x over previous
import functools

import jax
import jax.numpy as jnp
from jax import lax
from jax.experimental import pallas as pl
from jax.experimental.pallas import tpu as pltpu
from jax.experimental.pallas import tpu_sc as plsc

N_NODES_C = 10000
N_EDGES_C = 320000
D_FEAT_C = 128
N_HOPS_C = 3

HALF = D_FEAT_C // 2
N_SUBCORES = 16
EDGES_PER_TILE = N_EDGES_C // N_SUBCORES
CHUNK = 80
N_CHUNKS = EDGES_PER_TILE // CHUNK
W_SMALL = 624
W_LAST = N_NODES_C - 15 * W_SMALL


def _sc_body(embed_cat, row_hbm, col_hbm, trend_hbm, zeros_hbm,
             out1, out2, out3,
             acc, gbuf, rowbuf, colbuf, tbuf):
    c = lax.axis_index("c")
    s = lax.axis_index("s")
    row_off = c * N_NODES_C

    outs = [out1, out2, out3]

    def scale_group(g, _):
        tv16 = tbuf[pl.ds(g * 16, 16)]
        for l in range(16):
            i = g * 16 + l
            tv = jnp.full((16,), tv16[l], jnp.float32)
            for jj in range(HALF // 16):
                v = gbuf[i, pl.ds(jj * 16, 16)]
                gbuf[i, pl.ds(jj * 16, 16)] = v * tv
        return 0

    for h in range(N_HOPS_C):
        src = embed_cat if h == 0 else outs[h - 1]
        dst = outs[h]

        @pl.when(s < 15)
        def _():
            pltpu.sync_copy(zeros_hbm.at[pl.ds(0, W_SMALL)],
                            acc.at[pl.ds(s * W_SMALL, W_SMALL)])

        @pl.when(s == 15)
        def _():
            pltpu.sync_copy(zeros_hbm, acc.at[pl.ds(15 * W_SMALL, W_LAST)])

        plsc.subcore_barrier()

        def chunk_body(j, _):
            base = pl.multiple_of(s * EDGES_PER_TILE + j * CHUNK, CHUNK)
            pltpu.sync_copy(row_hbm.at[pl.ds(base, CHUNK)], rowbuf)
            pltpu.sync_copy(col_hbm.at[pl.ds(base, CHUNK)], colbuf)
            pltpu.sync_copy(trend_hbm.at[pl.ds(base, CHUNK)], tbuf)
            for v in range(CHUNK // 16):
                rowbuf[pl.ds(v * 16, 16)] = rowbuf[pl.ds(v * 16, 16)] + row_off
            pltpu.sync_copy(src.at[rowbuf], gbuf)
            lax.fori_loop(0, CHUNK // 16, scale_group, 0)
            pltpu.sync_copy(gbuf, acc.at[colbuf], add=True)
            return 0

        lax.fori_loop(0, N_CHUNKS, chunk_body, 0)
        plsc.subcore_barrier()

        @pl.when(s < 15)
        def _():
            pltpu.sync_copy(
                acc.at[pl.ds(s * W_SMALL, W_SMALL)],
                dst.at[pl.ds(row_off + s * W_SMALL, W_SMALL)])

        @pl.when(s == 15)
        def _():
            pltpu.sync_copy(
                acc.at[pl.ds(15 * W_SMALL, W_LAST)],
                dst.at[pl.ds(row_off + 15 * W_SMALL, W_LAST)])

        plsc.subcore_barrier()


@jax.jit
def _sc_call(embed_cat, row, col, trend, zeros):
    out_t = [jax.ShapeDtypeStruct((2 * N_NODES_C, HALF), jnp.float32)] * N_HOPS_C
    mesh = plsc.VectorSubcoreMesh(core_axis_name="c", subcore_axis_name="s")
    f = pl.kernel(
        _sc_body,
        out_type=out_t,
        mesh=mesh,
        compiler_params=pltpu.CompilerParams(use_tc_tiling_on_sc=False),
        scratch_types=[
            pltpu.VMEM_SHARED((N_NODES_C, HALF), jnp.float32),
            pltpu.VMEM((CHUNK, HALF), jnp.float32),
            pltpu.VMEM((CHUNK,), jnp.int32),
            pltpu.VMEM((CHUNK,), jnp.int32),
            pltpu.VMEM((CHUNK,), jnp.float32),
        ],
    )
    return f(embed_cat, row, col, trend, zeros)


def kernel(embed, adj_sp_norm, edge_index, edge_weight, trend):
    del adj_sp_norm, edge_weight
    row = edge_index[0].astype(jnp.int32)
    col = edge_index[1].astype(jnp.int32)
    embed_cat = jnp.concatenate([embed[:, :HALF], embed[:, HALF:]], axis=0)
    zeros = jnp.zeros((W_LAST, HALF), jnp.float32)
    out1, out2, out3 = _sc_call(embed_cat, row, col, trend, zeros)

    def unsplit(o):
        return jnp.concatenate([o[:N_NODES_C], o[N_NODES_C:]], axis=1)

    return jnp.stack(
        [embed, unsplit(out1), unsplit(out2), unsplit(out3)], axis=1)


# speedup vs baseline: 3.8828x; 2.3441x over previous
import jax
import jax.numpy as jnp
from jax import lax
from jax.experimental import pallas as pl
from jax.experimental.pallas import tpu as pltpu
from jax.experimental.pallas import tpu_sc as plsc

N_NODES_C = 10000
N_EDGES_C = 320000
D_FEAT_C = 128
N_HOPS_C = 3

HALF = D_FEAT_C // 2
N_SUBCORES = 16
EDGES_PER_TILE = N_EDGES_C // N_SUBCORES
CHUNK = 80
N_CHUNKS = EDGES_PER_TILE // CHUNK
NBUF = 5
W_SMALL = 624
W_LAST = N_NODES_C - 15 * W_SMALL


def _sc_body(embed_cat, row_hbm, col_hbm, trend_hbm, zeros_hbm,
             out1, out2, out3, cur,
             acc, gbufs, rowbufs, colbufs, tbufs,
             sem_idx, sem_g, sem_sc):
    c = lax.axis_index("c")
    s = lax.axis_index("s")
    row_off = c * N_NODES_C

    def ebase(q):
        return pl.multiple_of(s * EDGES_PER_TILE + q * CHUNK, 8)

    def idx_start(q, b):
        base = ebase(q)
        pltpu.make_async_copy(row_hbm.at[pl.ds(base, CHUNK)],
                              rowbufs.at[b], sem_idx.at[b]).start()
        pltpu.make_async_copy(col_hbm.at[pl.ds(base, CHUNK)],
                              colbufs.at[b], sem_idx.at[b]).start()
        pltpu.make_async_copy(trend_hbm.at[pl.ds(base, CHUNK)],
                              tbufs.at[b], sem_idx.at[b]).start()

    def idx_wait(b):
        pltpu.make_async_copy(row_hbm.at[pl.ds(0, CHUNK)],
                              rowbufs.at[b], sem_idx.at[b]).wait()
        pltpu.make_async_copy(col_hbm.at[pl.ds(0, CHUNK)],
                              colbufs.at[b], sem_idx.at[b]).wait()
        pltpu.make_async_copy(trend_hbm.at[pl.ds(0, CHUNK)],
                              tbufs.at[b], sem_idx.at[b]).wait()

    def rowfix(b):
        for v in range(CHUNK // 16):
            rowbufs[b, pl.ds(v * 16, 16)] = (
                rowbufs[b, pl.ds(v * 16, 16)] + row_off)

    def gather_start(b):
        pltpu.make_async_copy(cur.at[rowbufs.at[b]],
                              gbufs.at[b], sem_g.at[b]).start()

    def gather_wait(b):
        pltpu.make_async_copy(cur.at[rowbufs.at[b]],
                              gbufs.at[b], sem_g.at[b]).wait()

    def scat_start(b):
        pltpu.make_async_copy(gbufs.at[b], acc.at[colbufs.at[b]],
                              sem_sc.at[b]).start(add=True)

    def scat_wait(b):
        pltpu.make_async_copy(gbufs.at[b], acc.at[colbufs.at[b]],
                              sem_sc.at[b]).wait()

    def make_scale(b):
        def scale_group(g, _):
            tv16 = tbufs[b, pl.ds(g * 16, 16)]
            for l in range(16):
                i = g * 16 + l
                tv = jnp.full((16,), tv16[l], jnp.float32)
                for jj in range(HALF // 16):
                    v = gbufs[b, i, pl.ds(jj * 16, 16)]
                    gbufs[b, i, pl.ds(jj * 16, 16)] = v * tv
            return 0
        return scale_group

    scales = [make_scale(b) for b in range(NBUF)]

    @pl.when(s < 15)
    def _():
        pltpu.sync_copy(embed_cat.at[pl.ds(row_off + s * W_SMALL, W_SMALL)],
                        cur.at[pl.ds(row_off + s * W_SMALL, W_SMALL)])

    @pl.when(s == 15)
    def _():
        pltpu.sync_copy(embed_cat.at[pl.ds(row_off + 15 * W_SMALL, W_LAST)],
                        cur.at[pl.ds(row_off + 15 * W_SMALL, W_LAST)])

    def hop_body(h, _):
        @pl.when(s < 15)
        def _():
            pltpu.sync_copy(zeros_hbm.at[pl.ds(0, W_SMALL)],
                            acc.at[pl.ds(s * W_SMALL, W_SMALL)])

        @pl.when(s == 15)
        def _():
            pltpu.sync_copy(zeros_hbm, acc.at[pl.ds(15 * W_SMALL, W_LAST)])

        plsc.subcore_barrier()

        idx_start(0, 0)
        idx_start(1, 1)
        idx_wait(0)
        rowfix(0)
        gather_start(0)

        def outer(jo, _):
            for b in range(NBUF):
                q = jo * NBUF + b
                bp = (b + 2) % NBUF
                bn = (b + 1) % NBUF

                @pl.when(q + 2 < N_CHUNKS)
                def _():
                    @pl.when(q + 2 >= NBUF)
                    def _():
                        scat_wait(bp)
                    idx_start(q + 2, bp)

                @pl.when(q + 1 < N_CHUNKS)
                def _():
                    idx_wait(bn)
                    rowfix(bn)
                    gather_start(bn)

                gather_wait(b)
                lax.fori_loop(0, CHUNK // 16, scales[b], 0)
                scat_start(b)
            return 0

        lax.fori_loop(0, N_CHUNKS // NBUF, outer, 0)
        for b in range(NBUF):
            scat_wait(b)
        plsc.subcore_barrier()

        def write_out(dst):
            @pl.when(s < 15)
            def _():
                pltpu.sync_copy(
                    acc.at[pl.ds(s * W_SMALL, W_SMALL)],
                    dst.at[pl.ds(row_off + s * W_SMALL, W_SMALL)])

            @pl.when(s == 15)
            def _():
                pltpu.sync_copy(
                    acc.at[pl.ds(15 * W_SMALL, W_LAST)],
                    dst.at[pl.ds(row_off + 15 * W_SMALL, W_LAST)])

        write_out(cur)
        for hh, out in enumerate((out1, out2, out3)):
            @pl.when(h == hh)
            def _():
                write_out(out)
        plsc.subcore_barrier()
        return 0

    lax.fori_loop(0, N_HOPS_C, hop_body, 0)


@jax.jit
def _sc_call(embed_cat, row, col, trend, zeros):
    out_t = [jax.ShapeDtypeStruct((2 * N_NODES_C, HALF), jnp.float32)] * (
        N_HOPS_C + 1)
    mesh = plsc.VectorSubcoreMesh(core_axis_name="c", subcore_axis_name="s")
    f = pl.kernel(
        _sc_body,
        out_type=out_t,
        mesh=mesh,
        compiler_params=pltpu.CompilerParams(use_tc_tiling_on_sc=False),
        scratch_types=[
            pltpu.VMEM_SHARED((N_NODES_C, HALF), jnp.float32),
            pltpu.VMEM((NBUF, CHUNK, HALF), jnp.float32),
            pltpu.VMEM((NBUF, CHUNK), jnp.int32),
            pltpu.VMEM((NBUF, CHUNK), jnp.int32),
            pltpu.VMEM((NBUF, CHUNK), jnp.float32),
            pltpu.SemaphoreType.DMA((NBUF,)),
            pltpu.SemaphoreType.DMA((NBUF,)),
            pltpu.SemaphoreType.DMA((NBUF,)),
        ],
    )
    return f(embed_cat, row, col, trend, zeros)


def kernel(embed, adj_sp_norm, edge_index, edge_weight, trend):
    del adj_sp_norm, edge_weight
    row = edge_index[0].astype(jnp.int32)
    col = edge_index[1].astype(jnp.int32)
    embed_cat = jnp.concatenate([embed[:, :HALF], embed[:, HALF:]], axis=0)
    zeros = jnp.zeros((W_LAST, HALF), jnp.float32)
    out1, out2, out3, _ = _sc_call(embed_cat, row, col, trend, zeros)

    def unsplit(o):
        return jnp.concatenate([o[:N_NODES_C], o[N_NODES_C:]], axis=1)

    return jnp.stack(
        [embed, unsplit(out1), unsplit(out2), unsplit(out3)], axis=1)


# speedup vs baseline: 6.4592x; 1.6635x over previous
import jax
import jax.numpy as jnp
from jax import lax
from jax.experimental import pallas as pl
from jax.experimental.pallas import tpu as pltpu
from jax.experimental.pallas import tpu_sc as plsc

N_NODES_C = 10000
N_EDGES_C = 320000
D_FEAT_C = 128
N_HOPS_C = 3

HALF = D_FEAT_C // 2
N_SUBCORES = 16
EDGES_PER_TILE = N_EDGES_C // N_SUBCORES
CHUNK = 80
N_CHUNKS = EDGES_PER_TILE // CHUNK
NBUF = 5
W_SMALL = 624
W_LAST = N_NODES_C - 15 * W_SMALL


def _sc_body(embed_cat, row_hbm, col_hbm, trend_hbm, zeros_hbm,
             out1, out2, out3, cur,
             acc, gbufs, rowbufs, colbufs, tbufs,
             sem_idx, sem_g, sem_sc):
    c = lax.axis_index("c")
    s = lax.axis_index("s")
    row_off = c * N_NODES_C

    def ebase(q):
        return pl.multiple_of(s * EDGES_PER_TILE + q * CHUNK, 8)

    def idx_start(q, b):
        base = ebase(q)
        pltpu.make_async_copy(row_hbm.at[pl.ds(base, CHUNK)],
                              rowbufs.at[b], sem_idx.at[b]).start()
        pltpu.make_async_copy(col_hbm.at[pl.ds(base, CHUNK)],
                              colbufs.at[b], sem_idx.at[b]).start()
        pltpu.make_async_copy(trend_hbm.at[pl.ds(base, CHUNK)],
                              tbufs.at[b], sem_idx.at[b]).start()

    def idx_wait(b):
        pltpu.make_async_copy(row_hbm.at[pl.ds(0, CHUNK)],
                              rowbufs.at[b], sem_idx.at[b]).wait()
        pltpu.make_async_copy(col_hbm.at[pl.ds(0, CHUNK)],
                              colbufs.at[b], sem_idx.at[b]).wait()
        pltpu.make_async_copy(trend_hbm.at[pl.ds(0, CHUNK)],
                              tbufs.at[b], sem_idx.at[b]).wait()

    def rowfix(b):
        for v in range(CHUNK // 16):
            rowbufs[b, pl.ds(v * 16, 16)] = (
                rowbufs[b, pl.ds(v * 16, 16)] + row_off)

    def gather_start(b):
        pltpu.make_async_copy(cur.at[rowbufs.at[b]],
                              gbufs.at[b], sem_g.at[b]).start()

    def gather_wait(b):
        pltpu.make_async_copy(cur.at[rowbufs.at[b]],
                              gbufs.at[b], sem_g.at[b]).wait()

    def scat_start(b):
        pltpu.make_async_copy(gbufs.at[b], acc.at[colbufs.at[b]],
                              sem_sc.at[b]).start(add=True)

    def scat_wait(b):
        pltpu.make_async_copy(gbufs.at[b], acc.at[colbufs.at[b]],
                              sem_sc.at[b]).wait()

    def make_scale(b):
        def scale_group(g, _):
            tv16 = tbufs[b, pl.ds(g * 16, 16)]
            for sub in range(2):
                e0 = g * 16 + sub * 8
                tvs = [jnp.full((16,), tv16[sub * 8 + l], jnp.float32)
                       for l in range(8)]
                vs = [[gbufs[b, e0 + l, pl.ds(jj * 16, 16)]
                       for jj in range(HALF // 16)] for l in range(8)]
                for l in range(8):
                    for jj in range(HALF // 16):
                        gbufs[b, e0 + l, pl.ds(jj * 16, 16)] = (
                            vs[l][jj] * tvs[l])
            return 0
        return scale_group

    scales = [make_scale(b) for b in range(NBUF)]

    @pl.when(s < 15)
    def _():
        pltpu.sync_copy(embed_cat.at[pl.ds(row_off + s * W_SMALL, W_SMALL)],
                        cur.at[pl.ds(row_off + s * W_SMALL, W_SMALL)])

    @pl.when(s == 15)
    def _():
        pltpu.sync_copy(embed_cat.at[pl.ds(row_off + 15 * W_SMALL, W_LAST)],
                        cur.at[pl.ds(row_off + 15 * W_SMALL, W_LAST)])

    def hop_body(h, _):
        @pl.when(s < 15)
        def _():
            pltpu.sync_copy(zeros_hbm.at[pl.ds(0, W_SMALL)],
                            acc.at[pl.ds(s * W_SMALL, W_SMALL)])

        @pl.when(s == 15)
        def _():
            pltpu.sync_copy(zeros_hbm, acc.at[pl.ds(15 * W_SMALL, W_LAST)])

        plsc.subcore_barrier()

        idx_start(0, 0)
        idx_start(1, 1)
        idx_wait(0)
        rowfix(0)
        gather_start(0)

        def outer(jo, _):
            for b in range(NBUF):
                q = jo * NBUF + b
                bp = (b + 2) % NBUF
                bn = (b + 1) % NBUF

                @pl.when(q + 2 < N_CHUNKS)
                def _():
                    @pl.when(q + 2 >= NBUF)
                    def _():
                        scat_wait(bp)
                    idx_start(q + 2, bp)

                @pl.when(q + 1 < N_CHUNKS)
                def _():
                    idx_wait(bn)
                    rowfix(bn)
                    gather_start(bn)

                gather_wait(b)
                lax.fori_loop(0, CHUNK // 16, scales[b], 0)
                scat_start(b)
            return 0

        lax.fori_loop(0, N_CHUNKS // NBUF, outer, 0)
        for b in range(NBUF):
            scat_wait(b)
        plsc.subcore_barrier()

        def write_out(dst):
            @pl.when(s < 15)
            def _():
                pltpu.sync_copy(
                    acc.at[pl.ds(s * W_SMALL, W_SMALL)],
                    dst.at[pl.ds(row_off + s * W_SMALL, W_SMALL)])

            @pl.when(s == 15)
            def _():
                pltpu.sync_copy(
                    acc.at[pl.ds(15 * W_SMALL, W_LAST)],
                    dst.at[pl.ds(row_off + 15 * W_SMALL, W_LAST)])

        write_out(cur)
        for hh, out in enumerate((out1, out2, out3)):
            @pl.when(h == hh)
            def _():
                write_out(out)
        plsc.subcore_barrier()
        return 0

    lax.fori_loop(0, N_HOPS_C, hop_body, 0)


@jax.jit
def _sc_call(embed_cat, row, col, trend, zeros):
    out_t = [jax.ShapeDtypeStruct((2 * N_NODES_C, HALF), jnp.float32)] * (
        N_HOPS_C + 1)
    mesh = plsc.VectorSubcoreMesh(core_axis_name="c", subcore_axis_name="s")
    f = pl.kernel(
        _sc_body,
        out_type=out_t,
        mesh=mesh,
        compiler_params=pltpu.CompilerParams(use_tc_tiling_on_sc=False),
        scratch_types=[
            pltpu.VMEM_SHARED((N_NODES_C, HALF), jnp.float32),
            pltpu.VMEM((NBUF, CHUNK, HALF), jnp.float32),
            pltpu.VMEM((NBUF, CHUNK), jnp.int32),
            pltpu.VMEM((NBUF, CHUNK), jnp.int32),
            pltpu.VMEM((NBUF, CHUNK), jnp.float32),
            pltpu.SemaphoreType.DMA((NBUF,)),
            pltpu.SemaphoreType.DMA((NBUF,)),
            pltpu.SemaphoreType.DMA((NBUF,)),
        ],
    )
    return f(embed_cat, row, col, trend, zeros)


def kernel(embed, adj_sp_norm, edge_index, edge_weight, trend):
    del adj_sp_norm, edge_weight
    row = edge_index[0].astype(jnp.int32)
    col = edge_index[1].astype(jnp.int32)
    embed_cat = jnp.concatenate([embed[:, :HALF], embed[:, HALF:]], axis=0)
    zeros = jnp.zeros((W_LAST, HALF), jnp.float32)
    out1, out2, out3, _ = _sc_call(embed_cat, row, col, trend, zeros)

    def unsplit(o):
        return jnp.concatenate([o[:N_NODES_C], o[N_NODES_C:]], axis=1)

    return jnp.stack(
        [embed, unsplit(out1), unsplit(out2), unsplit(out3)], axis=1)
